# Initial kernel scaffold; baseline (speedup 1.0000x reference)
import jax
import jax.numpy as jnp
from jax.experimental import pallas as pl


def kernel(idx, dists, features_packed):
    raise NotImplementedError("write your pallas kernel here")


# speedup vs baseline: 14.1520x; 14.1520x over previous
import functools

import jax
import jax.numpy as jnp
from jax import lax
from jax.experimental import pallas as pl
from jax.experimental.pallas import tpu as pltpu
from jax.experimental.pallas import tpu_sc as plsc

_NC, _NS, _L = 2, 16, 16
_NW = _NC * _NS
_K = 8
_INV_R2 = 1.0 / (0.01 * 0.01)
_CHUNK = 1024


def _sc_render(idx_flat, d_flat, feats):
    npix = idx_flat.shape[0] // _K
    per_w = npix // _NW
    nchunk = per_w // _CHUNK
    mesh = plsc.VectorSubcoreMesh(core_axis_name="c", subcore_axis_name="s")

    @functools.partial(
        pl.kernel,
        out_type=jax.ShapeDtypeStruct((npix * 3,), jnp.float32),
        mesh=mesh,
        compiler_params=pltpu.CompilerParams(
            needs_layout_passes=False, use_tc_tiling_on_sc=False
        ),
        scratch_types=[
            pltpu.VMEM((_CHUNK * _K,), jnp.int32),
            pltpu.VMEM((_CHUNK * _K,), jnp.float32),
            pltpu.VMEM((_CHUNK * _K, 3), jnp.float32),
            pltpu.VMEM((_CHUNK * 3,), jnp.float32),
            pltpu.SemaphoreType.DMA,
        ],
    )
    def k(idx_hbm, d_hbm, feat_hbm, out_hbm, idx_v, d_v, rows_v, out_v, sem):
        wid = lax.axis_index("s") * _NC + lax.axis_index("c")
        iota = lax.iota(jnp.int32, _L)
        iota8 = iota * _K
        iota3 = iota * 3
        c0 = jnp.zeros((_L,), jnp.int32)
        c1 = jnp.full((_L,), 1, jnp.int32)
        c2 = jnp.full((_L,), 2, jnp.int32)

        @pl.loop(0, nchunk)
        def _chunk(ci):
            pixbase = (wid * nchunk + ci) * _CHUNK
            eb = pixbase * _K
            pltpu.sync_copy(idx_hbm.at[pl.ds(eb, _CHUNK * _K)], idx_v)
            pltpu.sync_copy(d_hbm.at[pl.ds(eb, _CHUNK * _K)], d_v)
            pltpu.async_copy(feat_hbm.at[idx_v], rows_v, sem).wait()

            @pl.loop(0, _CHUNK // _L)
            def _group(g):
                base = g * (_L * _K)
                den = jnp.full((_L,), 1e-10, jnp.float32)
                a0 = jnp.zeros((_L,), jnp.float32)
                a1 = jnp.zeros((_L,), jnp.float32)
                a2 = jnp.zeros((_L,), jnp.float32)
                for kk in range(_K):
                    ridx = iota8 + (base + kk)
                    d = plsc.load_gather(d_v, [ridx])
                    w = 1.0 - d * _INV_R2
                    den = den + w
                    f0 = plsc.load_gather(rows_v, [ridx, c0])
                    f1 = plsc.load_gather(rows_v, [ridx, c1])
                    f2 = plsc.load_gather(rows_v, [ridx, c2])
                    a0 = a0 + w * f0
                    a1 = a1 + w * f1
                    a2 = a2 + w * f2
                inv = 1.0 / den
                obase = iota3 + g * (_L * 3)
                plsc.store_scatter(out_v, [obase], a0 * inv)
                plsc.store_scatter(out_v, [obase + 1], a1 * inv)
                plsc.store_scatter(out_v, [obase + 2], a2 * inv)

            pltpu.sync_copy(out_v, out_hbm.at[pl.ds(pixbase * 3, _CHUNK * 3)])

    return k(idx_flat, d_flat, feats)


def kernel(idx, dists, features_packed):
    b, h, w, kk = idx.shape
    idx_flat = idx.reshape(-1).astype(jnp.int32)
    d_flat = dists.reshape(-1)
    out_flat = _sc_render(idx_flat, d_flat, features_packed)
    images = out_flat.reshape(b, h, w, 3)
    return images, idx, dists
